# Initial kernel scaffold; baseline (speedup 1.0000x reference)
import jax
import jax.numpy as jnp
from jax.experimental import pallas as pl


def kernel(substructure_indices, embedding_table):
    raise NotImplementedError("write your pallas kernel here")


# speedup vs baseline: 1.0226x; 1.0226x over previous
import functools

import jax
import jax.numpy as jnp
from jax import lax
from jax.experimental import pallas as pl
from jax.experimental.pallas import tpu as pltpu
from jax.experimental.pallas import tpu_sc as plsc

BATCH = 16384
HIST = 50
DIM = 32
TOTAL = BATCH * HIST
NUM_CORES = 2
NUM_SUBCORES = 16
NW = NUM_CORES * NUM_SUBCORES
PER_W = TOTAL // NW
CHUNK = 128
NCHUNK = PER_W // CHUNK

_mesh = plsc.VectorSubcoreMesh(core_axis_name="c", subcore_axis_name="s")


@functools.partial(
    pl.kernel,
    mesh=_mesh,
    out_type=jax.ShapeDtypeStruct((TOTAL, DIM), jnp.float32),
    scratch_types=[
        pltpu.VMEM((NCHUNK, CHUNK), jnp.int32),
        pltpu.VMEM((CHUNK, DIM), jnp.float32),
        pltpu.SemaphoreType.DMA,
    ],
    compiler_params=pltpu.CompilerParams(use_tc_tiling_on_sc=False),
)
def _gather_kernel(idx_hbm, table_hbm, out_hbm, idx_v, rows_v, sem):
    wid = lax.axis_index("s") * NUM_CORES + lax.axis_index("c")
    base = wid * PER_W
    pltpu.sync_copy(idx_hbm.at[wid], idx_v)

    def body(j, carry):
        pltpu.async_copy(table_hbm.at[idx_v.at[j]], rows_v, sem).wait()
        pltpu.sync_copy(rows_v, out_hbm.at[pl.ds(base + j * CHUNK, CHUNK)])
        return carry

    lax.fori_loop(0, NCHUNK, body, 0, unroll=False)


def kernel(substructure_indices, embedding_table):
    idx = substructure_indices.reshape(NW, NCHUNK, CHUNK)
    out = _gather_kernel(idx, embedding_table)
    return out.reshape(BATCH, HIST, DIM)


# speedup vs baseline: 1.1128x; 1.0883x over previous
import functools

import jax
import jax.numpy as jnp
from jax import lax
from jax.experimental import pallas as pl
from jax.experimental.pallas import tpu as pltpu
from jax.experimental.pallas import tpu_sc as plsc

BATCH = 16384
HIST = 50
DIM = 32
TOTAL = BATCH * HIST
NUM_CORES = 2
NUM_SUBCORES = 16
NW = NUM_CORES * NUM_SUBCORES
PER_W = TOTAL // NW
CHUNK = 128
NCHUNK = PER_W // CHUNK
NBUF = 8
PREF = 4
NGROUP = NCHUNK // NBUF

_mesh = plsc.VectorSubcoreMesh(core_axis_name="c", subcore_axis_name="s")


@functools.partial(
    pl.kernel,
    mesh=_mesh,
    out_type=jax.ShapeDtypeStruct((TOTAL, DIM), jnp.float32),
    scratch_types=[
        pltpu.VMEM((NCHUNK, CHUNK), jnp.int32),
        pltpu.VMEM((NBUF, CHUNK, DIM), jnp.float32),
        pltpu.SemaphoreType.DMA((NBUF,)),
        pltpu.SemaphoreType.DMA((NBUF,)),
    ],
    compiler_params=pltpu.CompilerParams(use_tc_tiling_on_sc=False),
)
def _gather_kernel(idx_hbm, table_hbm, out_hbm, idx_v, rows_v, gsem, osem):
    wid = lax.axis_index("s") * NUM_CORES + lax.axis_index("c")
    base = wid * PER_W
    pltpu.sync_copy(idx_hbm.at[wid], idx_v)

    def gather_desc(j, b):
        return pltpu.make_async_copy(
            table_hbm.at[idx_v.at[j]], rows_v.at[b], gsem.at[b])

    def out_desc(j, b):
        return pltpu.make_async_copy(
            rows_v.at[b],
            out_hbm.at[pl.ds(base + j * CHUNK, CHUNK)],
            osem.at[b])

    def step(j, b, wait_out_prev, issue_gather):
        bw = (b + PREF) % NBUF
        if wait_out_prev:
            out_desc(j + PREF - NBUF, bw).wait()
        if issue_gather:
            gather_desc(j + PREF, bw).start()
        gather_desc(j, b).wait()
        out_desc(j, b).start()

    for b in range(PREF):
        gather_desc(jnp.int32(b), b).start()

    for b in range(NBUF):
        step(jnp.int32(b), b, wait_out_prev=(b >= PREF), issue_gather=True)

    def group(g, carry):
        j0 = g * NBUF
        for b in range(NBUF):
            step(j0 + b, b, wait_out_prev=True, issue_gather=True)
        return carry

    lax.fori_loop(1, NGROUP - 1, group, 0, unroll=False)

    j0 = jnp.int32((NGROUP - 1) * NBUF)
    for b in range(NBUF):
        step(j0 + b, b, wait_out_prev=(b < PREF), issue_gather=(b < PREF))

    for b in range(NBUF):
        out_desc(jnp.int32(NCHUNK - NBUF + b), b).wait()


def kernel(substructure_indices, embedding_table):
    idx = substructure_indices.reshape(NW, NCHUNK, CHUNK)
    out = _gather_kernel(idx, embedding_table)
    return out.reshape(BATCH, HIST, DIM)


# speedup vs baseline: 1.6414x; 1.4750x over previous
import functools

import jax
import jax.numpy as jnp
from jax import lax
from jax.experimental import pallas as pl
from jax.experimental.pallas import tpu as pltpu
from jax.experimental.pallas import tpu_sc as plsc

BATCH = 16384
HIST = 50
DIM = 32
NUM_CORES = 2
NUM_SUBCORES = 16
NW = NUM_CORES * NUM_SUBCORES
BTILE = 128
NBT = BATCH // BTILE
UNITS = HIST * NBT
PER_W = UNITS // NW
NB = 4
NGROUP = PER_W // NB

_mesh = plsc.VectorSubcoreMesh(core_axis_name="c", subcore_axis_name="s")


@functools.partial(
    pl.kernel,
    mesh=_mesh,
    out_type=jax.ShapeDtypeStruct((HIST, DIM // 8, NBT, 8, BTILE), jnp.float32),
    scratch_types=[
        pltpu.VMEM((NB, BTILE), jnp.int32),
        pltpu.VMEM((NB, BTILE, DIM), jnp.float32),
        pltpu.VMEM((NB, DIM, BTILE), jnp.float32),
        pltpu.SemaphoreType.DMA((NB,)),
        pltpu.SemaphoreType.DMA((NB,)),
        pltpu.SemaphoreType.DMA((NB,)),
    ],
    compiler_params=pltpu.CompilerParams(
        use_tc_tiling_on_sc=False, needs_layout_passes=False),
)
def _gather_kernel(idx_hbm, table_hbm, out_hbm, idx_v, bufa, buft, isem, gsem, osem):
    wid = lax.axis_index("s") * NUM_CORES + lax.axis_index("c")
    u0 = wid * PER_W

    def hc(u):
        return u // NBT, lax.rem(u, NBT)

    def idx_desc(u, b):
        h, c = hc(u)
        return pltpu.make_async_copy(
            idx_hbm.at[h, pl.ds(c * BTILE, BTILE)], idx_v.at[b], isem.at[b])

    def gather_desc(b):
        return pltpu.make_async_copy(
            table_hbm.at[idx_v.at[b]], bufa.at[b], gsem.at[b])

    def out_desc(u, s, b):
        h, c = hc(u)
        return pltpu.make_async_copy(
            buft.at[b, pl.ds(s * 8, 8)], out_hbm.at[h, s, c], osem.at[b])

    def transpose(b):
        src = bufa.at[b]
        dst = buft.at[b]

        def body(eb, carry):
            rows = lax.iota(jnp.int32, 16) + jnp.full((16,), eb * 16, jnp.int32)
            for f in range(DIM):
                cols = jnp.full((16,), f, jnp.int32)
                vals = plsc.load_gather(src, [rows, cols])
                dst[f, pl.ds(eb * 16, 16)] = vals
            return carry

        lax.fori_loop(0, BTILE // 16, body, 0, unroll=False)

    def step(u, b, drain, gather_ahead, idx_ahead):
        bg = (b + 2) % NB
        if gather_ahead:
            idx_desc(u + 2, bg).wait()
            gather_desc(bg).start()
        if drain:
            for s in range(DIM // 8):
                out_desc(u - NB, s, b).wait()
        gather_desc(b).wait()
        if idx_ahead:
            idx_desc(u + NB, b).start()
        transpose(b)
        for s in range(DIM // 8):
            out_desc(u, s, b).start()

    for b in range(NB):
        idx_desc(u0 + b, b).start()
    for b in range(2):
        idx_desc(u0 + b, b).wait()
        gather_desc(b).start()

    for b in range(NB):
        step(u0 + b, b, drain=False, gather_ahead=True, idx_ahead=True)

    def group(g, carry):
        ug = u0 + g * NB
        for b in range(NB):
            step(ug + b, b, drain=True, gather_ahead=True, idx_ahead=True)
        return carry

    lax.fori_loop(1, NGROUP - 1, group, 0, unroll=False)

    ul = u0 + (NGROUP - 1) * NB
    for b in range(NB):
        step(ul + b, b, drain=True, gather_ahead=(b < 2), idx_ahead=False)

    for b in range(NB):
        for s in range(DIM // 8):
            out_desc(ul + b, s, b).wait()


def kernel(substructure_indices, embedding_table):
    idx_t = substructure_indices.T
    out5 = _gather_kernel(idx_t, embedding_table)
    return out5.transpose(2, 4, 0, 1, 3).reshape(BATCH, HIST, DIM)
